# Initial kernel scaffold; baseline (speedup 1.0000x reference)
import jax
import jax.numpy as jnp
from jax.experimental import pallas as pl


def kernel(x, edge_index, batch, W1, b1, W2, b2, Wih, Whh, bih, bhh, Wl, bl):
    raise NotImplementedError("write your pallas kernel here")


# speedup vs baseline: 11.5433x; 11.5433x over previous
import functools

import jax
import jax.numpy as jnp
from jax import lax
from jax.experimental import pallas as pl
from jax.experimental.pallas import tpu as pltpu
from jax.experimental.pallas import tpu_sc as plsc

N = 10000
E = 320000
D = 128
NC = 2
NS = 16
NW = NC * NS
EPW = E // NW
CH = 80
NCH = EPW // CH
NP = 10240
RPT = NP // NS
ZR = 128

_sc_mesh = plsc.VectorSubcoreMesh(core_axis_name="c", subcore_axis_name="s",
                                  num_cores=NC, num_subcores=NS)


@functools.partial(
    pl.kernel,
    out_type=jax.ShapeDtypeStruct((NW, 1, N), jnp.float32),
    mesh=_sc_mesh,
    scratch_types=[
        pltpu.VMEM((EPW,), jnp.int32),
        pltpu.VMEM((N,), jnp.float32),
    ],
    compiler_params=pltpu.CompilerParams(needs_layout_passes=False),
)
def _degree_kernel(src_hbm, out_hbm, idx_v, hist_v):
    cid = lax.axis_index("c")
    sid = lax.axis_index("s")
    wid = sid * NC + cid
    base = pl.multiple_of(wid * EPW, 8)
    pltpu.sync_copy(src_hbm.at[pl.ds(base, EPW)], idx_v)

    def zero_body(i, _):
        hist_v[pl.ds(i * 16, 16)] = jnp.zeros((16,), jnp.float32)
        return 0

    lax.fori_loop(0, N // 16, zero_body, 0)

    ones = jnp.ones((16,), jnp.float32)

    def acc_body(i, _):
        idx = idx_v[pl.ds(i * 16, 16)]
        plsc.addupdate_scatter(hist_v, [idx], ones)
        return 0

    lax.fori_loop(0, EPW // 16, acc_body, 0)

    pltpu.sync_copy(hist_v, out_hbm.at[wid, 0])


@functools.partial(
    pl.kernel,
    out_type=jax.ShapeDtypeStruct((NC, NP, D), jnp.float32),
    mesh=_sc_mesh,
    scratch_types=[
        pltpu.VMEM((CH,), jnp.int32),
        pltpu.VMEM((CH,), jnp.int32),
        pltpu.VMEM((CH, D), jnp.float32),
        pltpu.VMEM((ZR, D), jnp.float32),
        pltpu.VMEM_SHARED((NP, D), jnp.float32),
        pltpu.SemaphoreType.DMA,
    ],
    compiler_params=pltpu.CompilerParams(needs_layout_passes=False),
)
def _spmm_kernel(u_hbm, src_hbm, dst_hbm, out_hbm,
                 sidx_v, didx_v, rows_v, zero_v, acc_sh, sem):
    cid = lax.axis_index("c")
    sid = lax.axis_index("s")
    wid = sid * NC + cid
    base = wid * EPW

    def zbuf_body(i, _):
        for j in range(D // 16):
            zero_v[i, pl.ds(j * 16, 16)] = jnp.zeros((16,), jnp.float32)
        return 0

    lax.fori_loop(0, ZR, zbuf_body, 0)
    for i in range(RPT // ZR):
        pltpu.sync_copy(zero_v, acc_sh.at[pl.ds(sid * RPT + i * ZR, ZR)])
    plsc.subcore_barrier()

    def edge_body(j, _):
        off = pl.multiple_of(base + j * CH, 8)
        pltpu.sync_copy(src_hbm.at[pl.ds(off, CH)], sidx_v)
        pltpu.sync_copy(dst_hbm.at[pl.ds(off, CH)], didx_v)
        pltpu.async_copy(u_hbm.at[sidx_v], rows_v, sem).wait()
        pltpu.sync_copy(rows_v, acc_sh.at[didx_v], add=True)
        return 0

    lax.fori_loop(0, NCH, edge_body, 0)
    plsc.subcore_barrier()
    pltpu.sync_copy(acc_sh.at[pl.ds(sid * RPT, RPT)],
                    out_hbm.at[cid, pl.ds(sid * RPT, RPT)])


_RB = 2000


def _prep_body(degp_ref, x_ref, dinv_ref, u0_ref):
    deg = jnp.sum(degp_ref[...], axis=1, keepdims=True)
    dinv = jnp.where(deg > 0, lax.rsqrt(deg), 0.0)
    dinv_ref[...] = dinv
    u0_ref[...] = dinv * x_ref[...]


def _prep(degp, x):
    return pl.pallas_call(
        _prep_body,
        grid=(N // _RB,),
        in_specs=[
            pl.BlockSpec((_RB, NW), lambda i: (i, 0)),
            pl.BlockSpec((_RB, D), lambda i: (i, 0)),
        ],
        out_specs=[
            pl.BlockSpec((_RB, 1), lambda i: (i, 0)),
            pl.BlockSpec((_RB, D), lambda i: (i, 0)),
        ],
        out_shape=[
            jax.ShapeDtypeStruct((N, 1), jnp.float32),
            jax.ShapeDtypeStruct((N, D), jnp.float32),
        ],
    )(degp, x)


def _mid_body(p_ref, dinv_ref, tx1_ref, u1_ref):
    a = p_ref[0] + p_ref[1]
    dinv = dinv_ref[...]
    tx1 = -dinv * a
    tx1_ref[...] = tx1
    u1_ref[...] = dinv * tx1


def _mid(p, dinv):
    return pl.pallas_call(
        _mid_body,
        grid=(N // _RB,),
        in_specs=[
            pl.BlockSpec((NC, _RB, D), lambda i: (0, i, 0)),
            pl.BlockSpec((_RB, 1), lambda i: (i, 0)),
        ],
        out_specs=[
            pl.BlockSpec((_RB, D), lambda i: (i, 0)),
            pl.BlockSpec((_RB, D), lambda i: (i, 0)),
        ],
        out_shape=[
            jax.ShapeDtypeStruct((N, D), jnp.float32),
            jax.ShapeDtypeStruct((N, D), jnp.float32),
        ],
    )(p, dinv)


def _layer_out_body(q_ref, dinv_ref, xin_ref, tx1_ref,
                    w02_ref, w1_ref, w2_ref, b_ref,
                    h_ref, un_ref, *, with_unext):
    dinv = dinv_ref[...]
    tx2p = (-2.0 * dinv) * (q_ref[0] + q_ref[1])
    acc = jnp.dot(xin_ref[...], w02_ref[...],
                  preferred_element_type=jnp.float32)
    acc += jnp.dot(tx1_ref[...], w1_ref[...],
                   preferred_element_type=jnp.float32)
    acc += jnp.dot(tx2p, w2_ref[...], preferred_element_type=jnp.float32)
    h = jnp.maximum(acc + b_ref[...], 0.0)
    h_ref[...] = h
    if with_unext:
        un_ref[...] = dinv * h


def _layer_out(q, dinv, xin, tx1, w0, w1, w2, b, with_unext):
    body = functools.partial(_layer_out_body, with_unext=with_unext)
    n_out = 2 if with_unext else 1
    outs = pl.pallas_call(
        body if with_unext else
        (lambda q_ref, dinv_ref, xin_ref, tx1_ref, w02_ref, w1_ref, w2_ref,
                b_ref, h_ref:
         _layer_out_body(q_ref, dinv_ref, xin_ref, tx1_ref, w02_ref, w1_ref,
                         w2_ref, b_ref, h_ref, None, with_unext=False)),
        grid=(N // _RB,),
        in_specs=[
            pl.BlockSpec((NC, _RB, D), lambda i: (0, i, 0)),
            pl.BlockSpec((_RB, 1), lambda i: (i, 0)),
            pl.BlockSpec((_RB, D), lambda i: (i, 0)),
            pl.BlockSpec((_RB, D), lambda i: (i, 0)),
            pl.BlockSpec((D, D), lambda i: (0, 0)),
            pl.BlockSpec((D, D), lambda i: (0, 0)),
            pl.BlockSpec((D, D), lambda i: (0, 0)),
            pl.BlockSpec((1, D), lambda i: (0, 0)),
        ],
        out_specs=[pl.BlockSpec((_RB, D), lambda i: (i, 0))] * n_out,
        out_shape=[jax.ShapeDtypeStruct((N, D), jnp.float32)] * n_out,
    )(q, dinv, xin, tx1, w0 - w2, w1, w2, b.reshape(1, D))
    return outs if with_unext else (outs[0], None)


def _gi_body(h_ref, wihT_ref, bih_ref, gi_ref):
    gi_ref[...] = (jnp.dot(h_ref[...], wihT_ref[...],
                           preferred_element_type=jnp.float32)
                   + bih_ref[...])


def _gi(h, wihT, bih):
    return pl.pallas_call(
        _gi_body,
        grid=(N // _RB,),
        in_specs=[
            pl.BlockSpec((_RB, D), lambda i: (i, 0)),
            pl.BlockSpec((D, 3 * D), lambda i: (0, 0)),
            pl.BlockSpec((1, 3 * D), lambda i: (0, 0)),
        ],
        out_specs=pl.BlockSpec((_RB, 3 * D), lambda i: (i, 0)),
        out_shape=jax.ShapeDtypeStruct((N, 3 * D), jnp.float32),
    )(h, wihT, bih.reshape(1, 3 * D))


def _gru_body(gi_ref, whhT_ref, bhh_ref, wlT_ref, bl_ref, out_ref, ys_ref):
    whhT = whhT_ref[...]
    bhh = bhh_ref[...]

    def step(t, h):
        gi = gi_ref[pl.ds(t, 1), :]
        gh = jnp.dot(h, whhT, preferred_element_type=jnp.float32) + bhh
        r = jax.nn.sigmoid(gi[:, :D] + gh[:, :D])
        z = jax.nn.sigmoid(gi[:, D:2 * D] + gh[:, D:2 * D])
        n = jnp.tanh(gi[:, 2 * D:] + r * gh[:, 2 * D:])
        h_new = (1.0 - z) * n + z * h
        ys_ref[pl.ds(t, 1), :] = h_new
        return h_new

    lax.fori_loop(0, N, step, jnp.zeros((1, D), jnp.float32))
    out_ref[...] = (jnp.dot(ys_ref[...], wlT_ref[...],
                            preferred_element_type=jnp.float32)
                    + bl_ref[...])


def _gru(gi, whhT, bhh, wlT, bl):
    return pl.pallas_call(
        _gru_body,
        out_shape=jax.ShapeDtypeStruct((N, D), jnp.float32),
        scratch_shapes=[pltpu.VMEM((N, D), jnp.float32)],
    )(gi, whhT, bhh.reshape(1, 3 * D), wlT, bl.reshape(1, D))


def kernel(x, edge_index, batch, W1, b1, W2, b2, Wih, Whh, bih, bhh, Wl, bl):
    src = edge_index[0]
    dst = edge_index[1]

    degp = _degree_kernel(src)
    dinv, u0 = _prep(degp.reshape(NW, N).T, x)

    def cheb_layer(xin, uin, w, b, with_unext):
        p = _spmm_kernel(uin, src, dst)[:, :N, :]
        tx1, u1 = _mid(p, dinv)
        q = _spmm_kernel(u1, src, dst)[:, :N, :]
        return _layer_out(q, dinv, xin, tx1, w[0], w[1], w[2], b, with_unext)

    h1, u0b = cheb_layer(x, u0, W1, b1, True)
    h2, _ = cheb_layer(h1, u0b, W2, b2, False)

    gi = _gi(h2, Wih.T, bih)
    return _gru(gi, Whh.T, bhh, Wl.T, bl)


# speedup vs baseline: 16.6384x; 1.4414x over previous
import functools

import jax
import jax.numpy as jnp
from jax import lax
from jax.experimental import pallas as pl
from jax.experimental.pallas import tpu as pltpu
from jax.experimental.pallas import tpu_sc as plsc

N = 10000
E = 320000
D = 128
NC = 2
NS = 16
NW = NC * NS
EPW = E // NW
CH = 80
NCH = EPW // CH
NP = 10240
RPT = NP // NS
ZR = 128

_sc_mesh = plsc.VectorSubcoreMesh(core_axis_name="c", subcore_axis_name="s",
                                  num_cores=NC, num_subcores=NS)


@functools.partial(
    pl.kernel,
    out_type=jax.ShapeDtypeStruct((NW, 1, N), jnp.float32),
    mesh=_sc_mesh,
    scratch_types=[
        pltpu.VMEM((EPW,), jnp.int32),
        pltpu.VMEM((N,), jnp.float32),
    ],
    compiler_params=pltpu.CompilerParams(needs_layout_passes=False),
)
def _degree_kernel(src_hbm, out_hbm, idx_v, hist_v):
    cid = lax.axis_index("c")
    sid = lax.axis_index("s")
    wid = sid * NC + cid
    base = pl.multiple_of(wid * EPW, 8)
    pltpu.sync_copy(src_hbm.at[pl.ds(base, EPW)], idx_v)

    def zero_body(i, _):
        hist_v[pl.ds(i * 16, 16)] = jnp.zeros((16,), jnp.float32)
        return 0

    lax.fori_loop(0, N // 16, zero_body, 0)

    ones = jnp.ones((16,), jnp.float32)

    def acc_body(i, _):
        idx = idx_v[pl.ds(i * 16, 16)]
        plsc.addupdate_scatter(hist_v, [idx], ones)
        return 0

    lax.fori_loop(0, EPW // 16, acc_body, 0)

    pltpu.sync_copy(hist_v, out_hbm.at[wid, 0])


@functools.partial(
    pl.kernel,
    out_type=jax.ShapeDtypeStruct((NC, NP, D), jnp.float32),
    mesh=_sc_mesh,
    scratch_types=[
        pltpu.VMEM((EPW,), jnp.int32),
        pltpu.VMEM((NCH, CH), jnp.int32),
        pltpu.VMEM((2, CH, D), jnp.float32),
        pltpu.VMEM_SHARED((NP, D), jnp.float32),
        pltpu.SemaphoreType.DMA((2,)),
    ],
    compiler_params=pltpu.CompilerParams(needs_layout_passes=False),
)
def _spmm_kernel(u_hbm, src_hbm, dst_hbm, out_hbm,
                 sidx_v, didx_v, rows_v, acc_sh, gsem):
    cid = lax.axis_index("c")
    sid = lax.axis_index("s")
    wid = sid * NC + cid

    pltpu.sync_copy(src_hbm.at[pl.ds(pl.multiple_of(wid * EPW, 8), EPW)],
                    sidx_v)
    pltpu.sync_copy(dst_hbm.at[wid], didx_v)

    def zbuf_body(i, _):
        for b in range(2):
            for j in range(D // 16):
                rows_v[b, i, pl.ds(j * 16, 16)] = jnp.zeros((16,), jnp.float32)
        return 0

    lax.fori_loop(0, CH, zbuf_body, 0)
    for i in range(RPT // CH):
        pltpu.sync_copy(rows_v.at[0],
                        acc_sh.at[pl.ds(sid * RPT + i * CH, CH)])
    plsc.subcore_barrier()

    pltpu.async_copy(u_hbm.at[sidx_v.at[pl.ds(0, CH)]], rows_v.at[0],
                     gsem.at[0])

    def edge_body(j, _):
        b = lax.rem(j, 2)
        nb = lax.rem(j + 1, 2)

        @pl.when(j + 1 < NCH)
        def _():
            pltpu.async_copy(
                u_hbm.at[sidx_v.at[pl.ds((j + 1) * CH, CH)]],
                rows_v.at[nb], gsem.at[nb])

        pltpu.make_async_copy(u_hbm.at[sidx_v.at[pl.ds(j * CH, CH)]],
                              rows_v.at[b], gsem.at[b]).wait()
        pltpu.sync_copy(rows_v.at[b], acc_sh.at[didx_v.at[j]], add=True)
        return 0

    lax.fori_loop(0, NCH, edge_body, 0)
    plsc.subcore_barrier()
    pltpu.sync_copy(acc_sh.at[pl.ds(sid * RPT, RPT)],
                    out_hbm.at[cid, pl.ds(sid * RPT, RPT)])


_RB = 2000


def _prep_body(degp_ref, x_ref, dinv_ref, u0_ref):
    deg = jnp.sum(degp_ref[...], axis=1, keepdims=True)
    dinv = jnp.where(deg > 0, lax.rsqrt(deg), 0.0)
    dinv_ref[...] = dinv
    u0_ref[...] = dinv * x_ref[...]


def _prep(degp, x):
    return pl.pallas_call(
        _prep_body,
        grid=(N // _RB,),
        in_specs=[
            pl.BlockSpec((_RB, NW), lambda i: (i, 0)),
            pl.BlockSpec((_RB, D), lambda i: (i, 0)),
        ],
        out_specs=[
            pl.BlockSpec((_RB, 1), lambda i: (i, 0)),
            pl.BlockSpec((_RB, D), lambda i: (i, 0)),
        ],
        out_shape=[
            jax.ShapeDtypeStruct((N, 1), jnp.float32),
            jax.ShapeDtypeStruct((N, D), jnp.float32),
        ],
    )(degp, x)


def _mid_body(p_ref, dinv_ref, tx1_ref, u1_ref):
    a = p_ref[0] + p_ref[1]
    dinv = dinv_ref[...]
    tx1 = -dinv * a
    tx1_ref[...] = tx1
    u1_ref[...] = dinv * tx1


def _mid(p, dinv):
    return pl.pallas_call(
        _mid_body,
        grid=(N // _RB,),
        in_specs=[
            pl.BlockSpec((NC, _RB, D), lambda i: (0, i, 0)),
            pl.BlockSpec((_RB, 1), lambda i: (i, 0)),
        ],
        out_specs=[
            pl.BlockSpec((_RB, D), lambda i: (i, 0)),
            pl.BlockSpec((_RB, D), lambda i: (i, 0)),
        ],
        out_shape=[
            jax.ShapeDtypeStruct((N, D), jnp.float32),
            jax.ShapeDtypeStruct((N, D), jnp.float32),
        ],
    )(p, dinv)


def _layer_out_body(q_ref, dinv_ref, xin_ref, tx1_ref,
                    w02_ref, w1_ref, w2_ref, b_ref,
                    h_ref, un_ref, *, with_unext):
    dinv = dinv_ref[...]
    tx2p = (-2.0 * dinv) * (q_ref[0] + q_ref[1])
    acc = jnp.dot(xin_ref[...], w02_ref[...],
                  preferred_element_type=jnp.float32)
    acc += jnp.dot(tx1_ref[...], w1_ref[...],
                   preferred_element_type=jnp.float32)
    acc += jnp.dot(tx2p, w2_ref[...], preferred_element_type=jnp.float32)
    h = jnp.maximum(acc + b_ref[...], 0.0)
    h_ref[...] = h
    if with_unext:
        un_ref[...] = dinv * h


def _layer_out(q, dinv, xin, tx1, w0, w1, w2, b, with_unext):
    body = functools.partial(_layer_out_body, with_unext=with_unext)
    n_out = 2 if with_unext else 1
    outs = pl.pallas_call(
        body if with_unext else
        (lambda q_ref, dinv_ref, xin_ref, tx1_ref, w02_ref, w1_ref, w2_ref,
                b_ref, h_ref:
         _layer_out_body(q_ref, dinv_ref, xin_ref, tx1_ref, w02_ref, w1_ref,
                         w2_ref, b_ref, h_ref, None, with_unext=False)),
        grid=(N // _RB,),
        in_specs=[
            pl.BlockSpec((NC, _RB, D), lambda i: (0, i, 0)),
            pl.BlockSpec((_RB, 1), lambda i: (i, 0)),
            pl.BlockSpec((_RB, D), lambda i: (i, 0)),
            pl.BlockSpec((_RB, D), lambda i: (i, 0)),
            pl.BlockSpec((D, D), lambda i: (0, 0)),
            pl.BlockSpec((D, D), lambda i: (0, 0)),
            pl.BlockSpec((D, D), lambda i: (0, 0)),
            pl.BlockSpec((1, D), lambda i: (0, 0)),
        ],
        out_specs=[pl.BlockSpec((_RB, D), lambda i: (i, 0))] * n_out,
        out_shape=[jax.ShapeDtypeStruct((N, D), jnp.float32)] * n_out,
    )(q, dinv, xin, tx1, w0 - w2, w1, w2, b.reshape(1, D))
    return outs if with_unext else (outs[0], None)


def _gi_body(h_ref, wihT_ref, bih_ref, gi_ref):
    gi_ref[...] = (jnp.dot(h_ref[...], wihT_ref[...],
                           preferred_element_type=jnp.float32)
                   + bih_ref[...])


def _gi(h, wihT, bih):
    return pl.pallas_call(
        _gi_body,
        grid=(N // _RB,),
        in_specs=[
            pl.BlockSpec((_RB, D), lambda i: (i, 0)),
            pl.BlockSpec((D, 3 * D), lambda i: (0, 0)),
            pl.BlockSpec((1, 3 * D), lambda i: (0, 0)),
        ],
        out_specs=pl.BlockSpec((_RB, 3 * D), lambda i: (i, 0)),
        out_shape=jax.ShapeDtypeStruct((N, 3 * D), jnp.float32),
    )(h, wihT, bih.reshape(1, 3 * D))


def _gru_body(gi_ref, whhT_ref, bhh_ref, wlT_ref, bl_ref, out_ref, ys_ref):
    whhT = whhT_ref[...]
    bhh = bhh_ref[...]

    def step(t, h):
        gi = gi_ref[pl.ds(t, 1), :]
        gh = jnp.dot(h, whhT, preferred_element_type=jnp.float32) + bhh
        r = jax.nn.sigmoid(gi[:, :D] + gh[:, :D])
        z = jax.nn.sigmoid(gi[:, D:2 * D] + gh[:, D:2 * D])
        n = jnp.tanh(gi[:, 2 * D:] + r * gh[:, 2 * D:])
        h_new = (1.0 - z) * n + z * h
        ys_ref[pl.ds(t, 1), :] = h_new
        return h_new

    def step4(i, h):
        t = i * 4
        for k in range(4):
            h = step(t + k, h)
        return h

    lax.fori_loop(0, N // 4, step4, jnp.zeros((1, D), jnp.float32))
    out_ref[...] = (jnp.dot(ys_ref[...], wlT_ref[...],
                            preferred_element_type=jnp.float32)
                    + bl_ref[...])


def _gru(gi, whhT, bhh, wlT, bl):
    return pl.pallas_call(
        _gru_body,
        out_shape=jax.ShapeDtypeStruct((N, D), jnp.float32),
        scratch_shapes=[pltpu.VMEM((N, D), jnp.float32)],
    )(gi, whhT, bhh.reshape(1, 3 * D), wlT, bl.reshape(1, D))


def kernel(x, edge_index, batch, W1, b1, W2, b2, Wih, Whh, bih, bhh, Wl, bl):
    src = edge_index[0]
    dst = edge_index[1]

    degp = _degree_kernel(src)
    dinv, u0 = _prep(degp.reshape(NW, N).T, x)

    dst3 = dst.reshape(NW, NCH, CH)

    def cheb_layer(xin, uin, w, b, with_unext):
        p = _spmm_kernel(uin, src, dst3)[:, :N, :]
        tx1, u1 = _mid(p, dinv)
        q = _spmm_kernel(u1, src, dst3)[:, :N, :]
        return _layer_out(q, dinv, xin, tx1, w[0], w[1], w[2], b, with_unext)

    h1, u0b = cheb_layer(x, u0, W1, b1, True)
    h2, _ = cheb_layer(h1, u0b, W2, b2, False)

    gi = _gi(h2, Wih.T, bih)
    return _gru(gi, Whh.T, bhh, Wl.T, bl)


# speedup vs baseline: 18.6315x; 1.1198x over previous
import functools

import jax
import jax.numpy as jnp
from jax import lax
from jax.experimental import pallas as pl
from jax.experimental.pallas import tpu as pltpu
from jax.experimental.pallas import tpu_sc as plsc

N = 10000
E = 320000
D = 128
NC = 2
NS = 16
NW = NC * NS
EPW = E // NW
CH = 80
NCH = EPW // CH
NP = 10240
RPT = NP // NS
ZR = 128

_sc_mesh = plsc.VectorSubcoreMesh(core_axis_name="c", subcore_axis_name="s",
                                  num_cores=NC, num_subcores=NS)


@functools.partial(
    pl.kernel,
    out_type=jax.ShapeDtypeStruct((NW, 1, N), jnp.float32),
    mesh=_sc_mesh,
    scratch_types=[
        pltpu.VMEM((EPW,), jnp.int32),
        pltpu.VMEM((N,), jnp.float32),
    ],
    compiler_params=pltpu.CompilerParams(needs_layout_passes=False),
)
def _degree_kernel(src_hbm, out_hbm, idx_v, hist_v):
    cid = lax.axis_index("c")
    sid = lax.axis_index("s")
    wid = sid * NC + cid
    base = pl.multiple_of(wid * EPW, 8)
    pltpu.sync_copy(src_hbm.at[pl.ds(base, EPW)], idx_v)

    def zero_body(i, _):
        hist_v[pl.ds(i * 16, 16)] = jnp.zeros((16,), jnp.float32)
        return 0

    lax.fori_loop(0, N // 16, zero_body, 0)

    ones = jnp.ones((16,), jnp.float32)

    def acc_body(i, _):
        idx = idx_v[pl.ds(i * 16, 16)]
        plsc.addupdate_scatter(hist_v, [idx], ones)
        return 0

    lax.fori_loop(0, EPW // 16, acc_body, 0)

    pltpu.sync_copy(hist_v, out_hbm.at[wid, 0])


@functools.partial(
    pl.kernel,
    out_type=jax.ShapeDtypeStruct((NC, NP, D), jnp.float32),
    mesh=_sc_mesh,
    scratch_types=[
        pltpu.VMEM((EPW,), jnp.int32),
        pltpu.VMEM((NCH, CH), jnp.int32),
        pltpu.VMEM((2, CH, D), jnp.float32),
        pltpu.VMEM_SHARED((NP, D), jnp.float32),
        pltpu.SemaphoreType.DMA((2,)),
    ],
    compiler_params=pltpu.CompilerParams(needs_layout_passes=False),
)
def _spmm_kernel(u_hbm, src_hbm, dst_hbm, out_hbm,
                 sidx_v, didx_v, rows_v, acc_sh, gsem):
    cid = lax.axis_index("c")
    sid = lax.axis_index("s")
    wid = sid * NC + cid

    pltpu.sync_copy(src_hbm.at[pl.ds(pl.multiple_of(wid * EPW, 8), EPW)],
                    sidx_v)
    pltpu.sync_copy(dst_hbm.at[wid], didx_v)

    def zbuf_body(i, _):
        for b in range(2):
            for j in range(D // 16):
                rows_v[b, i, pl.ds(j * 16, 16)] = jnp.zeros((16,), jnp.float32)
        return 0

    lax.fori_loop(0, CH, zbuf_body, 0)
    for i in range(RPT // CH):
        pltpu.sync_copy(rows_v.at[0],
                        acc_sh.at[pl.ds(sid * RPT + i * CH, CH)])
    plsc.subcore_barrier()

    pltpu.async_copy(u_hbm.at[sidx_v.at[pl.ds(0, CH)]], rows_v.at[0],
                     gsem.at[0])

    def edge_body(j, _):
        b = lax.rem(j, 2)
        nb = lax.rem(j + 1, 2)

        @pl.when(j + 1 < NCH)
        def _():
            pltpu.async_copy(
                u_hbm.at[sidx_v.at[pl.ds((j + 1) * CH, CH)]],
                rows_v.at[nb], gsem.at[nb])

        pltpu.make_async_copy(u_hbm.at[sidx_v.at[pl.ds(j * CH, CH)]],
                              rows_v.at[b], gsem.at[b]).wait()
        pltpu.sync_copy(rows_v.at[b], acc_sh.at[didx_v.at[j]], add=True)
        return 0

    lax.fori_loop(0, NCH, edge_body, 0)
    plsc.subcore_barrier()
    pltpu.sync_copy(acc_sh.at[pl.ds(sid * RPT, RPT)],
                    out_hbm.at[cid, pl.ds(sid * RPT, RPT)])


_RB = 2000


def _prep_body(degp_ref, x_ref, dinv_ref, u0_ref):
    deg = jnp.sum(degp_ref[...], axis=1, keepdims=True)
    dinv = jnp.where(deg > 0, lax.rsqrt(deg), 0.0)
    dinv_ref[...] = dinv
    u0_ref[...] = dinv * x_ref[...]


def _prep(degp, x):
    return pl.pallas_call(
        _prep_body,
        grid=(N // _RB,),
        in_specs=[
            pl.BlockSpec((_RB, NW), lambda i: (i, 0)),
            pl.BlockSpec((_RB, D), lambda i: (i, 0)),
        ],
        out_specs=[
            pl.BlockSpec((_RB, 1), lambda i: (i, 0)),
            pl.BlockSpec((_RB, D), lambda i: (i, 0)),
        ],
        out_shape=[
            jax.ShapeDtypeStruct((N, 1), jnp.float32),
            jax.ShapeDtypeStruct((N, D), jnp.float32),
        ],
    )(degp, x)


def _mid_body(p_ref, dinv_ref, tx1_ref, u1_ref):
    a = p_ref[0] + p_ref[1]
    dinv = dinv_ref[...]
    tx1 = -dinv * a
    tx1_ref[...] = tx1
    u1_ref[...] = dinv * tx1


def _mid(p, dinv):
    return pl.pallas_call(
        _mid_body,
        grid=(N // _RB,),
        in_specs=[
            pl.BlockSpec((NC, _RB, D), lambda i: (0, i, 0)),
            pl.BlockSpec((_RB, 1), lambda i: (i, 0)),
        ],
        out_specs=[
            pl.BlockSpec((_RB, D), lambda i: (i, 0)),
            pl.BlockSpec((_RB, D), lambda i: (i, 0)),
        ],
        out_shape=[
            jax.ShapeDtypeStruct((N, D), jnp.float32),
            jax.ShapeDtypeStruct((N, D), jnp.float32),
        ],
    )(p, dinv)


def _layer_out_body(q_ref, dinv_ref, xin_ref, tx1_ref,
                    w02_ref, w1_ref, w2_ref, b_ref,
                    h_ref, un_ref, *, with_unext):
    dinv = dinv_ref[...]
    tx2p = (-2.0 * dinv) * (q_ref[0] + q_ref[1])
    acc = jnp.dot(xin_ref[...], w02_ref[...],
                  preferred_element_type=jnp.float32)
    acc += jnp.dot(tx1_ref[...], w1_ref[...],
                   preferred_element_type=jnp.float32)
    acc += jnp.dot(tx2p, w2_ref[...], preferred_element_type=jnp.float32)
    h = jnp.maximum(acc + b_ref[...], 0.0)
    h_ref[...] = h
    if with_unext:
        un_ref[...] = dinv * h


def _layer_out(q, dinv, xin, tx1, w0, w1, w2, b, with_unext):
    body = functools.partial(_layer_out_body, with_unext=with_unext)
    n_out = 2 if with_unext else 1
    outs = pl.pallas_call(
        body if with_unext else
        (lambda q_ref, dinv_ref, xin_ref, tx1_ref, w02_ref, w1_ref, w2_ref,
                b_ref, h_ref:
         _layer_out_body(q_ref, dinv_ref, xin_ref, tx1_ref, w02_ref, w1_ref,
                         w2_ref, b_ref, h_ref, None, with_unext=False)),
        grid=(N // _RB,),
        in_specs=[
            pl.BlockSpec((NC, _RB, D), lambda i: (0, i, 0)),
            pl.BlockSpec((_RB, 1), lambda i: (i, 0)),
            pl.BlockSpec((_RB, D), lambda i: (i, 0)),
            pl.BlockSpec((_RB, D), lambda i: (i, 0)),
            pl.BlockSpec((D, D), lambda i: (0, 0)),
            pl.BlockSpec((D, D), lambda i: (0, 0)),
            pl.BlockSpec((D, D), lambda i: (0, 0)),
            pl.BlockSpec((1, D), lambda i: (0, 0)),
        ],
        out_specs=[pl.BlockSpec((_RB, D), lambda i: (i, 0))] * n_out,
        out_shape=[jax.ShapeDtypeStruct((N, D), jnp.float32)] * n_out,
    )(q, dinv, xin, tx1, w0 - w2, w1, w2, b.reshape(1, D))
    return outs if with_unext else (outs[0], None)


def _gi_body(h_ref, wihT_ref, bih_ref, gi_ref):
    gi_ref[...] = (jnp.dot(h_ref[...], wihT_ref[...],
                           preferred_element_type=jnp.float32)
                   + bih_ref[...])


def _gi(h, wihT, bih):
    return pl.pallas_call(
        _gi_body,
        grid=(N // _RB,),
        in_specs=[
            pl.BlockSpec((_RB, D), lambda i: (i, 0)),
            pl.BlockSpec((D, 3 * D), lambda i: (0, 0)),
            pl.BlockSpec((1, 3 * D), lambda i: (0, 0)),
        ],
        out_specs=pl.BlockSpec((_RB, 3 * D), lambda i: (i, 0)),
        out_shape=jax.ShapeDtypeStruct((N, 3 * D), jnp.float32),
    )(h, wihT, bih.reshape(1, 3 * D))


def _tree_sum(xs):
    while len(xs) > 1:
        nxt = [xs[i] + xs[i + 1] for i in range(0, len(xs) - 1, 2)]
        if len(xs) % 2:
            nxt.append(xs[-1])
        xs = nxt
    return xs[0]


def _gru_body(gi_ref, w3_ref, bhh_ref, wlT_ref, bl_ref, out_ref, ys_ref):
    bhh = bhh_ref[...]

    def step(t, h):
        gi = gi_ref[pl.ds(t, 1), :]
        hcol = h.reshape(D, 1)
        parts = [hcol[8 * g:8 * g + 8, :] * w3_ref[g] for g in range(D // 8)]
        gh = jnp.sum(_tree_sum(parts), axis=0, keepdims=True) + bhh
        r = jax.nn.sigmoid(gi[:, :D] + gh[:, :D])
        z = jax.nn.sigmoid(gi[:, D:2 * D] + gh[:, D:2 * D])
        n = jnp.tanh(gi[:, 2 * D:] + r * gh[:, 2 * D:])
        h_new = (1.0 - z) * n + z * h
        ys_ref[pl.ds(t, 1), :] = h_new
        return h_new

    def step4(i, h):
        t = i * 4
        for k in range(4):
            h = step(t + k, h)
        return h

    lax.fori_loop(0, N // 4, step4, jnp.zeros((1, D), jnp.float32))
    out_ref[...] = (jnp.dot(ys_ref[...], wlT_ref[...],
                            preferred_element_type=jnp.float32)
                    + bl_ref[...])


def _gru(gi, whhT, bhh, wlT, bl):
    return pl.pallas_call(
        _gru_body,
        out_shape=jax.ShapeDtypeStruct((N, D), jnp.float32),
        scratch_shapes=[pltpu.VMEM((N, D), jnp.float32)],
    )(gi, whhT.reshape(D // 8, 8, 3 * D), bhh.reshape(1, 3 * D), wlT,
      bl.reshape(1, D))


def kernel(x, edge_index, batch, W1, b1, W2, b2, Wih, Whh, bih, bhh, Wl, bl):
    src = edge_index[0]
    dst = edge_index[1]

    degp = _degree_kernel(src)
    dinv, u0 = _prep(degp.reshape(NW, N).T, x)

    dst3 = dst.reshape(NW, NCH, CH)

    def cheb_layer(xin, uin, w, b, with_unext):
        p = _spmm_kernel(uin, src, dst3)[:, :N, :]
        tx1, u1 = _mid(p, dinv)
        q = _spmm_kernel(u1, src, dst3)[:, :N, :]
        return _layer_out(q, dinv, xin, tx1, w[0], w[1], w[2], b, with_unext)

    h1, u0b = cheb_layer(x, u0, W1, b1, True)
    h2, _ = cheb_layer(h1, u0b, W2, b2, False)

    gi = _gi(h2, Wih.T, bih)
    return _gru(gi, Whh.T, bhh, Wl.T, bl)


# speedup vs baseline: 19.0254x; 1.0211x over previous
import functools

import jax
import jax.numpy as jnp
from jax import lax
from jax.experimental import pallas as pl
from jax.experimental.pallas import tpu as pltpu
from jax.experimental.pallas import tpu_sc as plsc

N = 10000
E = 320000
D = 128
NC = 2
NS = 16
NW = NC * NS
EPW = E // NW
CH = 80
NCH = EPW // CH
NP = 10240
RPT = NP // NS
ZR = 128

_sc_mesh = plsc.VectorSubcoreMesh(core_axis_name="c", subcore_axis_name="s",
                                  num_cores=NC, num_subcores=NS)


@functools.partial(
    pl.kernel,
    out_type=jax.ShapeDtypeStruct((NW, 1, N), jnp.float32),
    mesh=_sc_mesh,
    scratch_types=[
        pltpu.VMEM((EPW,), jnp.int32),
        pltpu.VMEM((N,), jnp.float32),
    ],
    compiler_params=pltpu.CompilerParams(needs_layout_passes=False),
)
def _degree_kernel(src_hbm, out_hbm, idx_v, hist_v):
    cid = lax.axis_index("c")
    sid = lax.axis_index("s")
    wid = sid * NC + cid
    base = pl.multiple_of(wid * EPW, 8)
    pltpu.sync_copy(src_hbm.at[pl.ds(base, EPW)], idx_v)

    def zero_body(i, _):
        hist_v[pl.ds(i * 16, 16)] = jnp.zeros((16,), jnp.float32)
        return 0

    lax.fori_loop(0, N // 16, zero_body, 0)

    ones = jnp.ones((16,), jnp.float32)

    def acc_body(i, _):
        idx = idx_v[pl.ds(i * 16, 16)]
        plsc.addupdate_scatter(hist_v, [idx], ones)
        return 0

    lax.fori_loop(0, EPW // 16, acc_body, 0)

    pltpu.sync_copy(hist_v, out_hbm.at[wid, 0])


@functools.partial(
    pl.kernel,
    out_type=jax.ShapeDtypeStruct((NC, NP, D), jnp.float32),
    mesh=_sc_mesh,
    scratch_types=[
        pltpu.VMEM((EPW,), jnp.int32),
        pltpu.VMEM((NCH, CH), jnp.int32),
        pltpu.VMEM((2, CH, D), jnp.float32),
        pltpu.VMEM_SHARED((NP, D), jnp.float32),
        pltpu.SemaphoreType.DMA((2,)),
    ],
    compiler_params=pltpu.CompilerParams(needs_layout_passes=False),
)
def _spmm_kernel(u_hbm, src_hbm, dst_hbm, out_hbm,
                 sidx_v, didx_v, rows_v, acc_sh, gsem):
    cid = lax.axis_index("c")
    sid = lax.axis_index("s")
    wid = sid * NC + cid

    pltpu.sync_copy(src_hbm.at[pl.ds(pl.multiple_of(wid * EPW, 8), EPW)],
                    sidx_v)
    pltpu.sync_copy(dst_hbm.at[wid], didx_v)

    def zbuf_body(i, _):
        for b in range(2):
            for j in range(D // 16):
                rows_v[b, i, pl.ds(j * 16, 16)] = jnp.zeros((16,), jnp.float32)
        return 0

    lax.fori_loop(0, CH, zbuf_body, 0)
    for i in range(RPT // CH):
        pltpu.sync_copy(rows_v.at[0],
                        acc_sh.at[pl.ds(sid * RPT + i * CH, CH)])
    plsc.subcore_barrier()

    pltpu.async_copy(u_hbm.at[sidx_v.at[pl.ds(0, CH)]], rows_v.at[0],
                     gsem.at[0])

    def edge_body(j, _):
        b = lax.rem(j, 2)
        nb = lax.rem(j + 1, 2)

        @pl.when(j + 1 < NCH)
        def _():
            pltpu.async_copy(
                u_hbm.at[sidx_v.at[pl.ds((j + 1) * CH, CH)]],
                rows_v.at[nb], gsem.at[nb])

        pltpu.make_async_copy(u_hbm.at[sidx_v.at[pl.ds(j * CH, CH)]],
                              rows_v.at[b], gsem.at[b]).wait()
        pltpu.sync_copy(rows_v.at[b], acc_sh.at[didx_v.at[j]], add=True)
        return 0

    lax.fori_loop(0, NCH, edge_body, 0)
    plsc.subcore_barrier()
    pltpu.sync_copy(acc_sh.at[pl.ds(sid * RPT, RPT)],
                    out_hbm.at[cid, pl.ds(sid * RPT, RPT)])


_RB = 2000


def _prep_body(degp_ref, x_ref, dinv_ref, u0_ref):
    deg = jnp.sum(degp_ref[...], axis=1, keepdims=True)
    dinv = jnp.where(deg > 0, lax.rsqrt(deg), 0.0)
    dinv_ref[...] = dinv
    u0_ref[...] = dinv * x_ref[...]


def _prep(degp, x):
    return pl.pallas_call(
        _prep_body,
        grid=(N // _RB,),
        in_specs=[
            pl.BlockSpec((_RB, NW), lambda i: (i, 0)),
            pl.BlockSpec((_RB, D), lambda i: (i, 0)),
        ],
        out_specs=[
            pl.BlockSpec((_RB, 1), lambda i: (i, 0)),
            pl.BlockSpec((_RB, D), lambda i: (i, 0)),
        ],
        out_shape=[
            jax.ShapeDtypeStruct((N, 1), jnp.float32),
            jax.ShapeDtypeStruct((N, D), jnp.float32),
        ],
    )(degp, x)


def _mid_body(p_ref, dinv_ref, tx1_ref, u1_ref):
    a = p_ref[0] + p_ref[1]
    dinv = dinv_ref[...]
    tx1 = -dinv * a
    tx1_ref[...] = tx1
    u1_ref[...] = dinv * tx1


def _mid(p, dinv):
    return pl.pallas_call(
        _mid_body,
        grid=(N // _RB,),
        in_specs=[
            pl.BlockSpec((NC, _RB, D), lambda i: (0, i, 0)),
            pl.BlockSpec((_RB, 1), lambda i: (i, 0)),
        ],
        out_specs=[
            pl.BlockSpec((_RB, D), lambda i: (i, 0)),
            pl.BlockSpec((_RB, D), lambda i: (i, 0)),
        ],
        out_shape=[
            jax.ShapeDtypeStruct((N, D), jnp.float32),
            jax.ShapeDtypeStruct((N, D), jnp.float32),
        ],
    )(p, dinv)


def _layer_out_body(q_ref, dinv_ref, xin_ref, tx1_ref,
                    w02_ref, w1_ref, w2_ref, b_ref,
                    h_ref, un_ref, *, with_unext):
    dinv = dinv_ref[...]
    tx2p = (-2.0 * dinv) * (q_ref[0] + q_ref[1])
    acc = jnp.dot(xin_ref[...], w02_ref[...],
                  preferred_element_type=jnp.float32)
    acc += jnp.dot(tx1_ref[...], w1_ref[...],
                   preferred_element_type=jnp.float32)
    acc += jnp.dot(tx2p, w2_ref[...], preferred_element_type=jnp.float32)
    h = jnp.maximum(acc + b_ref[...], 0.0)
    h_ref[...] = h
    if with_unext:
        un_ref[...] = dinv * h


def _layer_out(q, dinv, xin, tx1, w0, w1, w2, b, with_unext):
    body = functools.partial(_layer_out_body, with_unext=with_unext)
    n_out = 2 if with_unext else 1
    outs = pl.pallas_call(
        body if with_unext else
        (lambda q_ref, dinv_ref, xin_ref, tx1_ref, w02_ref, w1_ref, w2_ref,
                b_ref, h_ref:
         _layer_out_body(q_ref, dinv_ref, xin_ref, tx1_ref, w02_ref, w1_ref,
                         w2_ref, b_ref, h_ref, None, with_unext=False)),
        grid=(N // _RB,),
        in_specs=[
            pl.BlockSpec((NC, _RB, D), lambda i: (0, i, 0)),
            pl.BlockSpec((_RB, 1), lambda i: (i, 0)),
            pl.BlockSpec((_RB, D), lambda i: (i, 0)),
            pl.BlockSpec((_RB, D), lambda i: (i, 0)),
            pl.BlockSpec((D, D), lambda i: (0, 0)),
            pl.BlockSpec((D, D), lambda i: (0, 0)),
            pl.BlockSpec((D, D), lambda i: (0, 0)),
            pl.BlockSpec((1, D), lambda i: (0, 0)),
        ],
        out_specs=[pl.BlockSpec((_RB, D), lambda i: (i, 0))] * n_out,
        out_shape=[jax.ShapeDtypeStruct((N, D), jnp.float32)] * n_out,
    )(q, dinv, xin, tx1, w0 - w2, w1, w2, b.reshape(1, D))
    return outs if with_unext else (outs[0], None)


def _gi_body(h_ref, wihT_ref, bih_ref, gi_ref):
    gi_ref[...] = (jnp.dot(h_ref[...], wihT_ref[...],
                           preferred_element_type=jnp.float32)
                   + bih_ref[...])


def _gi(h, wihT, bih):
    return pl.pallas_call(
        _gi_body,
        grid=(N // _RB,),
        in_specs=[
            pl.BlockSpec((_RB, D), lambda i: (i, 0)),
            pl.BlockSpec((D, 3 * D), lambda i: (0, 0)),
            pl.BlockSpec((1, 3 * D), lambda i: (0, 0)),
        ],
        out_specs=pl.BlockSpec((_RB, 3 * D), lambda i: (i, 0)),
        out_shape=jax.ShapeDtypeStruct((N, 3 * D), jnp.float32),
    )(h, wihT, bih.reshape(1, 3 * D))


def _tree_sum(xs):
    while len(xs) > 1:
        nxt = [xs[i] + xs[i + 1] for i in range(0, len(xs) - 1, 2)]
        if len(xs) % 2:
            nxt.append(xs[-1])
        xs = nxt
    return xs[0]


def _gru_body(gi_ref, w3_ref, bhh_ref, wlT_ref, bl_ref, out_ref, ys_ref):
    bhh = bhh_ref[...]

    def step(t, h):
        gi = gi_ref[pl.ds(t, 1), :]
        hcol = h.reshape(D, 1)
        parts = [hcol[8 * g:8 * g + 8, :] * w3_ref[g] for g in range(D // 8)]
        gh = jnp.sum(_tree_sum(parts), axis=0, keepdims=True) + bhh
        tr = jnp.tanh(0.5 * (gi[:, :D] + gh[:, :D]))
        tz = jnp.tanh(0.5 * (gi[:, D:2 * D] + gh[:, D:2 * D]))
        hn = gh[:, 2 * D:]
        n = jnp.tanh(gi[:, 2 * D:] + 0.5 * hn + 0.5 * tr * hn)
        h_new = 0.5 * ((n + h) + tz * (h - n))
        ys_ref[pl.ds(t, 1), :] = h_new
        return h_new

    def step8(i, h):
        t = i * 8
        for k in range(8):
            h = step(t + k, h)
        return h

    lax.fori_loop(0, N // 8, step8, jnp.zeros((1, D), jnp.float32))
    out_ref[...] = (jnp.dot(ys_ref[...], wlT_ref[...],
                            preferred_element_type=jnp.float32)
                    + bl_ref[...])


def _gru(gi, whhT, bhh, wlT, bl):
    return pl.pallas_call(
        _gru_body,
        out_shape=jax.ShapeDtypeStruct((N, D), jnp.float32),
        scratch_shapes=[pltpu.VMEM((N, D), jnp.float32)],
    )(gi, whhT.reshape(D // 8, 8, 3 * D), bhh.reshape(1, 3 * D), wlT,
      bl.reshape(1, D))


def kernel(x, edge_index, batch, W1, b1, W2, b2, Wih, Whh, bih, bhh, Wl, bl):
    src = edge_index[0]
    dst = edge_index[1]

    degp = _degree_kernel(src)
    dinv, u0 = _prep(degp.reshape(NW, N).T, x)

    dst3 = dst.reshape(NW, NCH, CH)

    def cheb_layer(xin, uin, w, b, with_unext):
        p = _spmm_kernel(uin, src, dst3)[:, :N, :]
        tx1, u1 = _mid(p, dinv)
        q = _spmm_kernel(u1, src, dst3)[:, :N, :]
        return _layer_out(q, dinv, xin, tx1, w[0], w[1], w[2], b, with_unext)

    h1, u0b = cheb_layer(x, u0, W1, b1, True)
    h2, _ = cheb_layer(h1, u0b, W2, b2, False)

    gi = _gi(h2, Wih.T, bih)
    return _gru(gi, Whh.T, bhh, Wl.T, bl)


# speedup vs baseline: 19.0315x; 1.0003x over previous
import functools

import jax
import jax.numpy as jnp
from jax import lax
from jax.experimental import pallas as pl
from jax.experimental.pallas import tpu as pltpu
from jax.experimental.pallas import tpu_sc as plsc

N = 10000
E = 320000
D = 128
NC = 2
NS = 16
NW = NC * NS
EPW = E // NW
CH = 80
NCH = EPW // CH
NP = 10240
RPT = NP // NS
ZR = 128

_sc_mesh = plsc.VectorSubcoreMesh(core_axis_name="c", subcore_axis_name="s",
                                  num_cores=NC, num_subcores=NS)


@functools.partial(
    pl.kernel,
    out_type=jax.ShapeDtypeStruct((NW, 1, N), jnp.float32),
    mesh=_sc_mesh,
    scratch_types=[
        pltpu.VMEM((EPW,), jnp.int32),
        pltpu.VMEM((N,), jnp.float32),
    ],
    compiler_params=pltpu.CompilerParams(needs_layout_passes=False),
)
def _degree_kernel(src_hbm, out_hbm, idx_v, hist_v):
    cid = lax.axis_index("c")
    sid = lax.axis_index("s")
    wid = sid * NC + cid
    base = pl.multiple_of(wid * EPW, 8)
    pltpu.sync_copy(src_hbm.at[pl.ds(base, EPW)], idx_v)

    def zero_body(i, _):
        hist_v[pl.ds(i * 16, 16)] = jnp.zeros((16,), jnp.float32)
        return 0

    lax.fori_loop(0, N // 16, zero_body, 0)

    ones = jnp.ones((16,), jnp.float32)

    def acc_body(i, _):
        idx = idx_v[pl.ds(i * 16, 16)]
        plsc.addupdate_scatter(hist_v, [idx], ones)
        return 0

    lax.fori_loop(0, EPW // 16, acc_body, 0)

    pltpu.sync_copy(hist_v, out_hbm.at[wid, 0])


@functools.partial(
    pl.kernel,
    out_type=jax.ShapeDtypeStruct((NC, NP, D), jnp.float32),
    mesh=_sc_mesh,
    scratch_types=[
        pltpu.VMEM((EPW,), jnp.int32),
        pltpu.VMEM((NCH, CH), jnp.int32),
        pltpu.VMEM((2, CH, D), jnp.float32),
        pltpu.VMEM_SHARED((NP, D), jnp.float32),
        pltpu.SemaphoreType.DMA((2,)),
        pltpu.SemaphoreType.DMA((2,)),
    ],
    compiler_params=pltpu.CompilerParams(needs_layout_passes=False),
)
def _spmm_kernel(u_hbm, src_hbm, dst_hbm, out_hbm,
                 sidx_v, didx_v, rows_v, acc_sh, gsem, ssem):
    cid = lax.axis_index("c")
    sid = lax.axis_index("s")
    wid = sid * NC + cid

    pltpu.sync_copy(src_hbm.at[pl.ds(pl.multiple_of(wid * EPW, 8), EPW)],
                    sidx_v)
    pltpu.sync_copy(dst_hbm.at[wid], didx_v)

    def zbuf_body(i, _):
        for b in range(2):
            for j in range(D // 16):
                rows_v[b, i, pl.ds(j * 16, 16)] = jnp.zeros((16,), jnp.float32)
        return 0

    lax.fori_loop(0, CH, zbuf_body, 0)
    for i in range(RPT // CH):
        pltpu.sync_copy(rows_v.at[0],
                        acc_sh.at[pl.ds(sid * RPT + i * CH, CH)])
    plsc.subcore_barrier()

    pltpu.async_copy(u_hbm.at[sidx_v.at[pl.ds(0, CH)]], rows_v.at[0],
                     gsem.at[0])

    def edge_body(j, _):
        b = lax.rem(j, 2)
        nb = lax.rem(j + 1, 2)

        @pl.when(j + 1 < NCH)
        def _():
            @pl.when(j >= 1)
            def _():
                pltpu.make_async_copy(rows_v.at[nb],
                                      acc_sh.at[didx_v.at[j - 1]],
                                      ssem.at[nb]).wait()

            pltpu.async_copy(
                u_hbm.at[sidx_v.at[pl.ds((j + 1) * CH, CH)]],
                rows_v.at[nb], gsem.at[nb])

        pltpu.make_async_copy(u_hbm.at[sidx_v.at[pl.ds(j * CH, CH)]],
                              rows_v.at[b], gsem.at[b]).wait()
        pltpu.async_copy(rows_v.at[b], acc_sh.at[didx_v.at[j]], ssem.at[b],
                         add=True)
        return 0

    lax.fori_loop(0, NCH, edge_body, 0)
    pltpu.make_async_copy(rows_v.at[1], acc_sh.at[didx_v.at[NCH - 2]],
                          ssem.at[1]).wait()
    pltpu.make_async_copy(rows_v.at[0], acc_sh.at[didx_v.at[NCH - 1]],
                          ssem.at[0]).wait()
    plsc.subcore_barrier()
    pltpu.sync_copy(acc_sh.at[pl.ds(sid * RPT, RPT)],
                    out_hbm.at[cid, pl.ds(sid * RPT, RPT)])


_RB = 2000


def _prep_body(degp_ref, x_ref, dinv_ref, u0_ref):
    deg = jnp.sum(degp_ref[...], axis=1, keepdims=True)
    dinv = jnp.where(deg > 0, lax.rsqrt(deg), 0.0)
    dinv_ref[...] = dinv
    u0_ref[...] = dinv * x_ref[...]


def _prep(degp, x):
    return pl.pallas_call(
        _prep_body,
        grid=(N // _RB,),
        in_specs=[
            pl.BlockSpec((_RB, NW), lambda i: (i, 0)),
            pl.BlockSpec((_RB, D), lambda i: (i, 0)),
        ],
        out_specs=[
            pl.BlockSpec((_RB, 1), lambda i: (i, 0)),
            pl.BlockSpec((_RB, D), lambda i: (i, 0)),
        ],
        out_shape=[
            jax.ShapeDtypeStruct((N, 1), jnp.float32),
            jax.ShapeDtypeStruct((N, D), jnp.float32),
        ],
    )(degp, x)


def _mid_body(p_ref, dinv_ref, tx1_ref, u1_ref):
    a = p_ref[0] + p_ref[1]
    dinv = dinv_ref[...]
    tx1 = -dinv * a
    tx1_ref[...] = tx1
    u1_ref[...] = dinv * tx1


def _mid(p, dinv):
    return pl.pallas_call(
        _mid_body,
        grid=(N // _RB,),
        in_specs=[
            pl.BlockSpec((NC, _RB, D), lambda i: (0, i, 0)),
            pl.BlockSpec((_RB, 1), lambda i: (i, 0)),
        ],
        out_specs=[
            pl.BlockSpec((_RB, D), lambda i: (i, 0)),
            pl.BlockSpec((_RB, D), lambda i: (i, 0)),
        ],
        out_shape=[
            jax.ShapeDtypeStruct((N, D), jnp.float32),
            jax.ShapeDtypeStruct((N, D), jnp.float32),
        ],
    )(p, dinv)


def _layer_out_body(q_ref, dinv_ref, xin_ref, tx1_ref,
                    w02_ref, w1_ref, w2_ref, b_ref,
                    h_ref, un_ref, *, with_unext):
    dinv = dinv_ref[...]
    tx2p = (-2.0 * dinv) * (q_ref[0] + q_ref[1])
    acc = jnp.dot(xin_ref[...], w02_ref[...],
                  preferred_element_type=jnp.float32)
    acc += jnp.dot(tx1_ref[...], w1_ref[...],
                   preferred_element_type=jnp.float32)
    acc += jnp.dot(tx2p, w2_ref[...], preferred_element_type=jnp.float32)
    h = jnp.maximum(acc + b_ref[...], 0.0)
    h_ref[...] = h
    if with_unext:
        un_ref[...] = dinv * h


def _layer_out(q, dinv, xin, tx1, w0, w1, w2, b, with_unext):
    body = functools.partial(_layer_out_body, with_unext=with_unext)
    n_out = 2 if with_unext else 1
    outs = pl.pallas_call(
        body if with_unext else
        (lambda q_ref, dinv_ref, xin_ref, tx1_ref, w02_ref, w1_ref, w2_ref,
                b_ref, h_ref:
         _layer_out_body(q_ref, dinv_ref, xin_ref, tx1_ref, w02_ref, w1_ref,
                         w2_ref, b_ref, h_ref, None, with_unext=False)),
        grid=(N // _RB,),
        in_specs=[
            pl.BlockSpec((NC, _RB, D), lambda i: (0, i, 0)),
            pl.BlockSpec((_RB, 1), lambda i: (i, 0)),
            pl.BlockSpec((_RB, D), lambda i: (i, 0)),
            pl.BlockSpec((_RB, D), lambda i: (i, 0)),
            pl.BlockSpec((D, D), lambda i: (0, 0)),
            pl.BlockSpec((D, D), lambda i: (0, 0)),
            pl.BlockSpec((D, D), lambda i: (0, 0)),
            pl.BlockSpec((1, D), lambda i: (0, 0)),
        ],
        out_specs=[pl.BlockSpec((_RB, D), lambda i: (i, 0))] * n_out,
        out_shape=[jax.ShapeDtypeStruct((N, D), jnp.float32)] * n_out,
    )(q, dinv, xin, tx1, w0 - w2, w1, w2, b.reshape(1, D))
    return outs if with_unext else (outs[0], None)


def _gi_body(h_ref, wihT_ref, bih_ref, gi_ref):
    gi_ref[...] = (jnp.dot(h_ref[...], wihT_ref[...],
                           preferred_element_type=jnp.float32)
                   + bih_ref[...])


def _gi(h, wihT, bih):
    return pl.pallas_call(
        _gi_body,
        grid=(N // _RB,),
        in_specs=[
            pl.BlockSpec((_RB, D), lambda i: (i, 0)),
            pl.BlockSpec((D, 3 * D), lambda i: (0, 0)),
            pl.BlockSpec((1, 3 * D), lambda i: (0, 0)),
        ],
        out_specs=pl.BlockSpec((_RB, 3 * D), lambda i: (i, 0)),
        out_shape=jax.ShapeDtypeStruct((N, 3 * D), jnp.float32),
    )(h, wihT, bih.reshape(1, 3 * D))


def _tree_sum(xs):
    while len(xs) > 1:
        nxt = [xs[i] + xs[i + 1] for i in range(0, len(xs) - 1, 2)]
        if len(xs) % 2:
            nxt.append(xs[-1])
        xs = nxt
    return xs[0]


def _gru_body(gi_ref, w3_ref, bhh_ref, wlT_ref, bl_ref, out_ref, ys_ref):
    bhh = bhh_ref[...]
    w3 = w3_ref[...]

    def step(t, h):
        gi = gi_ref[pl.ds(t, 1), :]
        hcol = h.reshape(D, 1)
        parts = [hcol[8 * g:8 * g + 8, :] * w3[g] for g in range(D // 8)]
        gh = jnp.sum(_tree_sum(parts), axis=0, keepdims=True) + bhh
        tr = jnp.tanh(0.5 * (gi[:, :D] + gh[:, :D]))
        tz = jnp.tanh(0.5 * (gi[:, D:2 * D] + gh[:, D:2 * D]))
        hn = gh[:, 2 * D:]
        n = jnp.tanh(gi[:, 2 * D:] + 0.5 * hn + 0.5 * tr * hn)
        h_new = 0.5 * ((n + h) + tz * (h - n))
        ys_ref[pl.ds(t, 1), :] = h_new
        return h_new

    def step8(i, h):
        t = i * 8
        for k in range(8):
            h = step(t + k, h)
        return h

    lax.fori_loop(0, N // 8, step8, jnp.zeros((1, D), jnp.float32))
    out_ref[...] = (jnp.dot(ys_ref[...], wlT_ref[...],
                            preferred_element_type=jnp.float32)
                    + bl_ref[...])


def _gru(gi, whhT, bhh, wlT, bl):
    return pl.pallas_call(
        _gru_body,
        out_shape=jax.ShapeDtypeStruct((N, D), jnp.float32),
        scratch_shapes=[pltpu.VMEM((N, D), jnp.float32)],
    )(gi, whhT.reshape(D // 8, 8, 3 * D), bhh.reshape(1, 3 * D), wlT,
      bl.reshape(1, D))


def kernel(x, edge_index, batch, W1, b1, W2, b2, Wih, Whh, bih, bhh, Wl, bl):
    src = edge_index[0]
    dst = edge_index[1]

    degp = _degree_kernel(src)
    dinv, u0 = _prep(degp.reshape(NW, N).T, x)

    dst3 = dst.reshape(NW, NCH, CH)

    def cheb_layer(xin, uin, w, b, with_unext):
        p = _spmm_kernel(uin, src, dst3)[:, :N, :]
        tx1, u1 = _mid(p, dinv)
        q = _spmm_kernel(u1, src, dst3)[:, :N, :]
        return _layer_out(q, dinv, xin, tx1, w[0], w[1], w[2], b, with_unext)

    h1, u0b = cheb_layer(x, u0, W1, b1, True)
    h2, _ = cheb_layer(h1, u0b, W2, b2, False)

    gi = _gi(h2, Wih.T, bih)
    return _gru(gi, Whh.T, bhh, Wl.T, bl)
